# baseline (speedup 1.0000x reference)
import jax
import jax.numpy as jnp
from jax.experimental import pallas as pl
from jax.experimental.pallas import tpu as pltpu


def _sdconv_block(lr_ref, li_ref, xc_ref, wr_ref, wi_ref, b_ref, out_ref):
    i = pl.program_id(1)
    xc = xc_ref[...].astype(jnp.bfloat16)
    yr = jnp.dot(lr_ref[...].astype(jnp.bfloat16), xc,
                 preferred_element_type=jnp.float32)
    yi = jnp.dot(li_ref[...].astype(jnp.bfloat16), xc,
                 preferred_element_type=jnp.float32)
    contrib = jnp.dot(yr.astype(jnp.bfloat16), wr_ref[0],
                      preferred_element_type=jnp.float32)
    contrib = contrib + jnp.dot(yi.astype(jnp.bfloat16), wi_ref[0],
                                preferred_element_type=jnp.float32)

    @pl.when(i == 0)
    def _():
        out_ref[...] = jnp.broadcast_to(b_ref[...], out_ref.shape) + contrib

    @pl.when(i != 0)
    def _():
        out_ref[...] = out_ref[...] + contrib


def kernel(X_real, X_imag, L_norm_real, L_norm_imag, weight, bias):
    N, D = X_real.shape
    Kp1, _, D_out = weight.shape

    xc = jnp.concatenate([X_real, X_imag], axis=1)
    z = jnp.zeros_like(weight)
    wr = jnp.concatenate(
        [jnp.concatenate([weight, z], axis=2),
         jnp.concatenate([z, weight], axis=2)], axis=1).astype(jnp.bfloat16)
    wi = jnp.concatenate(
        [jnp.concatenate([z, weight], axis=2),
         jnp.concatenate([-weight, z], axis=2)], axis=1).astype(jnp.bfloat16)
    b2 = jnp.concatenate([bias, bias], axis=1)

    lr2 = L_norm_real.reshape(Kp1 * N, N)
    li2 = L_norm_imag.reshape(Kp1 * N, N)

    BM = 256
    R = N // BM
    out = pl.pallas_call(
        _sdconv_block,
        grid=(R, Kp1),
        in_specs=[
            pl.BlockSpec((BM, N), lambda r, i: (i * R + r, 0)),
            pl.BlockSpec((BM, N), lambda r, i: (i * R + r, 0)),
            pl.BlockSpec((N, 2 * D), lambda r, i: (0, 0)),
            pl.BlockSpec((1, 2 * D, 2 * D_out), lambda r, i: (i, 0, 0)),
            pl.BlockSpec((1, 2 * D, 2 * D_out), lambda r, i: (i, 0, 0)),
            pl.BlockSpec((1, 2 * D_out), lambda r, i: (0, 0)),
        ],
        out_specs=pl.BlockSpec((BM, 2 * D_out), lambda r, i: (r, 0)),
        out_shape=jax.ShapeDtypeStruct((N, 2 * D_out), jnp.float32),
        compiler_params=pltpu.CompilerParams(
            dimension_semantics=(pltpu.PARALLEL, pltpu.ARBITRARY)),
    )(lr2, li2, xc, wr, wi, b2)

    real = out[:, :D_out]
    imag = out[:, D_out:]
    return (real, imag, L_norm_real, L_norm_imag)
